# baseline (speedup 1.0000x reference)
import functools

import jax
import jax.numpy as jnp
from jax import lax
from jax.experimental import pallas as pl
from jax.experimental.pallas import tpu as pltpu
from jax.experimental.pallas import tpu_sc as plsc

_EMBEDDING_DIM = 128
_BATCH = 16384
_ROWS = 1000

_info = plsc.get_sparse_core_info()
_NC, _NS = _info.num_cores, _info.num_subcores
_NW = _NC * _NS
_B_PER_W = _BATCH // _NW
_CHUNK = 256
_NCHUNK = _B_PER_W // _CHUNK
_STAGE_BIG, _N_BIG = 64, 13
_STAGE_SMALL = 56


@jax.jit
def _gather(table, idx):
  mesh = plsc.VectorSubcoreMesh(core_axis_name="c", subcore_axis_name="s")

  @functools.partial(
      pl.kernel,
      mesh=mesh,
      out_type=jax.ShapeDtypeStruct((_BATCH, _EMBEDDING_DIM), jnp.float32),
      scratch_types=[
          pltpu.VMEM_SHARED((_ROWS, _EMBEDDING_DIM), jnp.float32),
          pltpu.VMEM((_B_PER_W,), jnp.int32),
          pltpu.VMEM((_NCHUNK, _CHUNK, _EMBEDDING_DIM), jnp.float32),
          pltpu.SemaphoreType.DMA,
          pltpu.SemaphoreType.DMA,
          pltpu.SemaphoreType.DMA,
      ],
  )
  def k(table_hbm, idx_hbm, out_hbm, table_sh, idx_v, rows_v, gsem, ssem, tsem):
    sid = lax.axis_index("s")
    wid = sid * _NC + lax.axis_index("c")
    out_base = wid * _B_PER_W
    idx_cp = pltpu.async_copy(idx_hbm.at[pl.ds(out_base, _B_PER_W)], idx_v, gsem)

    @pl.when(sid < _N_BIG)
    def _stage_big():
      pltpu.async_copy(
          table_hbm.at[pl.ds(sid * _STAGE_BIG, _STAGE_BIG)],
          table_sh.at[pl.ds(sid * _STAGE_BIG, _STAGE_BIG)],
          tsem,
      ).wait()

    @pl.when(sid >= _N_BIG)
    def _stage_small():
      base = _N_BIG * _STAGE_BIG + (sid - _N_BIG) * _STAGE_SMALL
      pltpu.async_copy(
          table_hbm.at[pl.ds(base, _STAGE_SMALL)],
          table_sh.at[pl.ds(base, _STAGE_SMALL)],
          tsem,
      ).wait()

    idx_cp.wait()
    plsc.subcore_barrier()
    gathers = [
        pltpu.async_copy(
            table_sh.at[idx_v.at[pl.ds(c * _CHUNK, _CHUNK)]], rows_v.at[c], gsem
        )
        for c in range(_NCHUNK)
    ]
    stores = []
    for c in range(_NCHUNK):
      gathers[c].wait()
      stores.append(
          pltpu.async_copy(
              rows_v.at[c],
              out_hbm.at[pl.ds(out_base + c * _CHUNK, _CHUNK)],
              ssem,
          )
      )
    for st in stores:
      st.wait()

  return k(table, idx)


def kernel(t, pos_encoding):
  table = pos_encoding.reshape(pos_encoding.shape[0], _EMBEDDING_DIM)
  out = _gather(table, t.astype(jnp.int32))
  return out.reshape(_BATCH, 1, _EMBEDDING_DIM)


# speedup vs baseline: 1.0118x; 1.0118x over previous
import functools

import jax
import jax.numpy as jnp
from jax import lax
from jax.experimental import pallas as pl
from jax.experimental.pallas import tpu as pltpu
from jax.experimental.pallas import tpu_sc as plsc

_EMBEDDING_DIM = 128
_BATCH = 16384
_ROWS = 1000

_info = plsc.get_sparse_core_info()
_NC, _NS = _info.num_cores, _info.num_subcores
_NW = _NC * _NS
_B_PER_W = _BATCH // _NW
_CHUNK = 64
_NCHUNK = _B_PER_W // _CHUNK
_STAGE_BIG, _N_BIG = 64, 13
_STAGE_SMALL = 56


@jax.jit
def _gather(table, idx):
  mesh = plsc.VectorSubcoreMesh(core_axis_name="c", subcore_axis_name="s")

  @functools.partial(
      pl.kernel,
      mesh=mesh,
      out_type=jax.ShapeDtypeStruct((_BATCH, _EMBEDDING_DIM), jnp.float32),
      scratch_types=[
          pltpu.VMEM_SHARED((_ROWS, _EMBEDDING_DIM), jnp.float32),
          pltpu.VMEM((_B_PER_W,), jnp.int32),
          pltpu.VMEM((_NCHUNK, _CHUNK, _EMBEDDING_DIM), jnp.float32),
          pltpu.SemaphoreType.DMA,
          pltpu.SemaphoreType.DMA,
          pltpu.SemaphoreType.DMA,
      ],
  )
  def k(table_hbm, idx_hbm, out_hbm, table_sh, idx_v, rows_v, gsem, ssem, tsem):
    sid = lax.axis_index("s")
    wid = sid * _NC + lax.axis_index("c")
    out_base = wid * _B_PER_W
    idx_cp = pltpu.async_copy(idx_hbm.at[pl.ds(out_base, _B_PER_W)], idx_v, gsem)

    @pl.when(sid < _N_BIG)
    def _stage_big():
      pltpu.async_copy(
          table_hbm.at[pl.ds(sid * _STAGE_BIG, _STAGE_BIG)],
          table_sh.at[pl.ds(sid * _STAGE_BIG, _STAGE_BIG)],
          tsem,
      ).wait()

    @pl.when(sid >= _N_BIG)
    def _stage_small():
      base = _N_BIG * _STAGE_BIG + (sid - _N_BIG) * _STAGE_SMALL
      pltpu.async_copy(
          table_hbm.at[pl.ds(base, _STAGE_SMALL)],
          table_sh.at[pl.ds(base, _STAGE_SMALL)],
          tsem,
      ).wait()

    idx_cp.wait()
    plsc.subcore_barrier()
    gathers = [
        pltpu.async_copy(
            table_sh.at[idx_v.at[pl.ds(c * _CHUNK, _CHUNK)]], rows_v.at[c], gsem
        )
        for c in range(_NCHUNK)
    ]
    stores = []
    for c in range(_NCHUNK):
      gathers[c].wait()
      stores.append(
          pltpu.async_copy(
              rows_v.at[c],
              out_hbm.at[pl.ds(out_base + c * _CHUNK, _CHUNK)],
              ssem,
          )
      )
    for st in stores:
      st.wait()

  return k(table, idx)


def kernel(t, pos_encoding):
  table = pos_encoding.reshape(pos_encoding.shape[0], _EMBEDDING_DIM)
  out = _gather(table, t.astype(jnp.int32))
  return out.reshape(_BATCH, 1, _EMBEDDING_DIM)


# speedup vs baseline: 1.0198x; 1.0079x over previous
import functools

import jax
import jax.numpy as jnp
from jax import lax
from jax.experimental import pallas as pl
from jax.experimental.pallas import tpu as pltpu
from jax.experimental.pallas import tpu_sc as plsc

_EMBEDDING_DIM = 128
_BATCH = 16384
_ROWS = 1000

_info = plsc.get_sparse_core_info()
_NC, _NS = _info.num_cores, _info.num_subcores
_NW = _NC * _NS
_B_PER_W = _BATCH // _NW
_CHUNK = 128
_NCHUNK = _B_PER_W // _CHUNK
_STAGE_BIG, _N_BIG = 64, 13
_STAGE_SMALL = 56


@jax.jit
def _gather(table, idx):
  mesh = plsc.VectorSubcoreMesh(core_axis_name="c", subcore_axis_name="s")

  @functools.partial(
      pl.kernel,
      mesh=mesh,
      out_type=jax.ShapeDtypeStruct((_BATCH, _EMBEDDING_DIM), jnp.float32),
      scratch_types=[
          pltpu.VMEM_SHARED((_ROWS, _EMBEDDING_DIM), jnp.float32),
          pltpu.VMEM((_B_PER_W,), jnp.int32),
          pltpu.VMEM((_NCHUNK, _CHUNK, _EMBEDDING_DIM), jnp.float32),
          pltpu.SemaphoreType.DMA,
          pltpu.SemaphoreType.DMA,
          pltpu.SemaphoreType.DMA,
      ],
  )
  def k(table_hbm, idx_hbm, out_hbm, table_sh, idx_v, rows_v, gsem, ssem, tsem):
    sid = lax.axis_index("s")
    wid = sid * _NC + lax.axis_index("c")
    out_base = wid * _B_PER_W
    idx_cp = pltpu.async_copy(idx_hbm.at[pl.ds(out_base, _B_PER_W)], idx_v, gsem)

    @pl.when(sid < _N_BIG)
    def _stage_big():
      pltpu.async_copy(
          table_hbm.at[pl.ds(sid * _STAGE_BIG, _STAGE_BIG)],
          table_sh.at[pl.ds(sid * _STAGE_BIG, _STAGE_BIG)],
          tsem,
      ).wait()

    @pl.when(sid >= _N_BIG)
    def _stage_small():
      base = _N_BIG * _STAGE_BIG + (sid - _N_BIG) * _STAGE_SMALL
      pltpu.async_copy(
          table_hbm.at[pl.ds(base, _STAGE_SMALL)],
          table_sh.at[pl.ds(base, _STAGE_SMALL)],
          tsem,
      ).wait()

    idx_cp.wait()
    plsc.subcore_barrier()
    gathers = [
        pltpu.async_copy(
            table_sh.at[idx_v.at[pl.ds(c * _CHUNK, _CHUNK)]], rows_v.at[c], gsem
        )
        for c in range(_NCHUNK)
    ]
    stores = []
    for c in range(_NCHUNK):
      gathers[c].wait()
      stores.append(
          pltpu.async_copy(
              rows_v.at[c],
              out_hbm.at[pl.ds(out_base + c * _CHUNK, _CHUNK)],
              ssem,
          )
      )
    for st in stores:
      st.wait()

  return k(table, idx)


def kernel(t, pos_encoding):
  table = pos_encoding.reshape(pos_encoding.shape[0], _EMBEDDING_DIM)
  out = _gather(table, t.astype(jnp.int32))
  return out.reshape(_BATCH, 1, _EMBEDDING_DIM)
